# baseline (speedup 1.0000x reference)
import functools

import jax
import jax.numpy as jnp
from jax import lax
from jax.experimental import pallas as pl
from jax.experimental.pallas import tpu as pltpu
from jax.experimental.pallas import tpu_sc as plsc

N = 10000
E = 320000
D = 128
NP = 10240
DUMMY = N
NTILES = 32
CHUNK = 112
EPT_CHUNKS = 93
EPT = CHUNK * EPT_CHUNKS
E_PAD = EPT * NTILES
ROWS_PER_TILE = NP // 16


def _prep_body(x_ref, w_ref, as_ref, ad_ref, h_ref, asrc_ref, adst_ref):
    h = lax.dot_general(x_ref[...], w_ref[...], (((1,), (1,)), ((), ())),
                        preferred_element_type=jnp.float32)
    h_ref[...] = h
    asrc_ref[...] = lax.dot_general(h, as_ref[...], (((1,), (1,)), ((), ())),
                                    preferred_element_type=jnp.float32)
    adst_ref[...] = lax.dot_general(h, ad_ref[...], (((1,), (1,)), ((), ())),
                                    preferred_element_type=jnp.float32)


TCB = 1024


def _prep(x_pad, W, a_src, a_dst):
    return pl.pallas_call(
        _prep_body,
        grid=(NP // TCB,),
        in_specs=[
            pl.BlockSpec((TCB, D), lambda i: (i, 0)),
            pl.BlockSpec((D, D), lambda i: (0, 0)),
            pl.BlockSpec((1, D), lambda i: (0, 0)),
            pl.BlockSpec((1, D), lambda i: (0, 0)),
        ],
        out_specs=[
            pl.BlockSpec((TCB, D), lambda i: (i, 0)),
            pl.BlockSpec((TCB, 1), lambda i: (i, 0)),
            pl.BlockSpec((TCB, 1), lambda i: (i, 0)),
        ],
        out_shape=[
            jax.ShapeDtypeStruct((NP, D), jnp.float32),
            jax.ShapeDtypeStruct((NP, 1), jnp.float32),
            jax.ShapeDtypeStruct((NP, 1), jnp.float32),
        ],
    )(x_pad, W, a_src, a_dst)


def _combine_x(p0_ref, p1_ref, d0_ref, d1_ref, b_ref):
    den = d0_ref[...] + d1_ref[...] + 1e-16
    return (p0_ref[...] + p1_ref[...]) / den + b_ref[...]


def _combine_prep_body(p0_ref, p1_ref, d0_ref, d1_ref, b_ref,
                       w_ref, as_ref, ad_ref, h_ref, asrc_ref, adst_ref):
    y = _combine_x(p0_ref, p1_ref, d0_ref, d1_ref, b_ref)
    x2 = jnp.where(y > 0, y, jnp.exp(y) - 1.0)
    h = lax.dot_general(x2, w_ref[...], (((1,), (1,)), ((), ())),
                        preferred_element_type=jnp.float32)
    h_ref[...] = h
    asrc_ref[...] = lax.dot_general(h, as_ref[...], (((1,), (1,)), ((), ())),
                                    preferred_element_type=jnp.float32)
    adst_ref[...] = lax.dot_general(h, ad_ref[...], (((1,), (1,)), ((), ())),
                                    preferred_element_type=jnp.float32)


def _combine_prep(p0, p1, d0, d1, bias, W, a_src, a_dst):
    return pl.pallas_call(
        _combine_prep_body,
        grid=(NP // TCB,),
        in_specs=[
            pl.BlockSpec((TCB, D), lambda i: (i, 0)),
            pl.BlockSpec((TCB, D), lambda i: (i, 0)),
            pl.BlockSpec((TCB, 1), lambda i: (i, 0)),
            pl.BlockSpec((TCB, 1), lambda i: (i, 0)),
            pl.BlockSpec((1, D), lambda i: (0, 0)),
            pl.BlockSpec((D, D), lambda i: (0, 0)),
            pl.BlockSpec((1, D), lambda i: (0, 0)),
            pl.BlockSpec((1, D), lambda i: (0, 0)),
        ],
        out_specs=[
            pl.BlockSpec((TCB, D), lambda i: (i, 0)),
            pl.BlockSpec((TCB, 1), lambda i: (i, 0)),
            pl.BlockSpec((TCB, 1), lambda i: (i, 0)),
        ],
        out_shape=[
            jax.ShapeDtypeStruct((NP, D), jnp.float32),
            jax.ShapeDtypeStruct((NP, 1), jnp.float32),
            jax.ShapeDtypeStruct((NP, 1), jnp.float32),
        ],
    )(p0, p1, d0, d1, bias, W, a_src, a_dst)


def _combine_body(p0_ref, p1_ref, d0_ref, d1_ref, b_ref, y_ref):
    y_ref[...] = _combine_x(p0_ref, p1_ref, d0_ref, d1_ref, b_ref)


def _combine(p0, p1, d0, d1, bias):
    return pl.pallas_call(
        _combine_body,
        grid=(NP // TCB,),
        in_specs=[
            pl.BlockSpec((TCB, D), lambda i: (i, 0)),
            pl.BlockSpec((TCB, D), lambda i: (i, 0)),
            pl.BlockSpec((TCB, 1), lambda i: (i, 0)),
            pl.BlockSpec((TCB, 1), lambda i: (i, 0)),
            pl.BlockSpec((1, D), lambda i: (0, 0)),
        ],
        out_specs=pl.BlockSpec((TCB, D), lambda i: (i, 0)),
        out_shape=jax.ShapeDtypeStruct((NP, D), jnp.float32),
    )(p0, p1, d0, d1, bias)


def _edge_body(src_hbm, dst_hbm, asrc_hbm, adst_hbm, h_hbm,
               out0, out1, den0, den1,
               si0, si1, si2, di0, di1, di2,
               av0, av1, av2, bv0, bv1, bv2,
               ev0, ev1, ev2, r0, r1, r2,
               acc, dacc,
               qi0, qi1, qi2, qa0, qa1, qa2,
               qg0, qg1, qg2, qs0, qs1, qs2, qd0, qd1, qd2):
    sidx = (si0, si1, si2)
    didx = (di0, di1, di2)
    asv = (av0, av1, av2)
    adv = (bv0, bv1, bv2)
    eev = (ev0, ev1, ev2)
    rows = (r0, r1, r2)
    semi = (qi0, qi1, qi2)
    sema = (qa0, qa1, qa2)
    semg = (qg0, qg1, qg2)
    sems = (qs0, qs1, qs2)
    semd = (qd0, qd1, qd2)
    c = lax.axis_index("c")
    s = lax.axis_index("s")
    tile = c * 16 + s
    e0 = tile * EPT

    def _start_idx(g, b):
        base = pl.multiple_of(e0 + g * CHUNK, 8)
        pltpu.async_copy(src_hbm.at[pl.ds(base, CHUNK)], sidx[b], semi[b])
        pltpu.async_copy(dst_hbm.at[pl.ds(base, CHUNK)], didx[b], semi[b])

    def _drain(dummy_hbm, buf, sem):
        pltpu.make_async_copy(dummy_hbm, buf, sem).wait()

    def _drain_idx(b):
        _drain(src_hbm.at[pl.ds(0, CHUNK)], sidx[b], semi[b])
        _drain(src_hbm.at[pl.ds(0, CHUNK)], didx[b], semi[b])

    def _start_gathers(b):
        pltpu.async_copy(h_hbm.at[sidx[b]], rows[b], semg[b])
        pltpu.async_copy(asrc_hbm.at[sidx[b]], asv[b], sema[b])
        pltpu.async_copy(adst_hbm.at[didx[b]], adv[b], sema[b])

    def _drain_gathers(b):
        _drain(h_hbm.at[pl.ds(0, CHUNK)], rows[b], semg[b])
        _drain(asrc_hbm.at[pl.ds(0, CHUNK)], asv[b], sema[b])
        _drain(asrc_hbm.at[pl.ds(0, CHUNK)], adv[b], sema[b])

    def _drain_scatter(b):
        _drain(h_hbm.at[pl.ds(0, CHUNK)], rows[b], sems[b])
        _drain(asrc_hbm.at[pl.ds(0, CHUNK)], eev[b], semd[b])

    z16 = jnp.zeros((16,), jnp.float32)

    def _zrow(i, carry):
        for q in range(D // 16):
            r0[i, pl.ds(q * 16, 16)] = z16
        return carry

    lax.fori_loop(0, CHUNK, _zrow, 0)
    row0_ = s * ROWS_PER_TILE
    nfull = ROWS_PER_TILE // CHUNK
    rem = ROWS_PER_TILE - nfull * CHUNK
    for bb in range(nfull):
        pltpu.sync_copy(r0, acc.at[pl.ds(row0_ + bb * CHUNK, CHUNK)])
    pltpu.sync_copy(r0.at[pl.ds(0, rem)],
                    acc.at[pl.ds(row0_ + nfull * CHUNK, rem)])
    for bb in range(ROWS_PER_TILE // D):
        pltpu.sync_copy(r0.at[0], dacc.at[pl.ds(row0_ + bb * D, D)])

    _start_idx(0, 0)
    _start_idx(1, 1)
    _drain_idx(0)
    _start_gathers(0)
    plsc.subcore_barrier()

    def _step(st, carry):
        for b in range(3):
            gg = st * 3 + b
            bp = (b + 2) % 3
            bn = (b + 1) % 3

            @pl.when(gg >= 1)
            def _():
                _drain_scatter(bp)

            @pl.when(gg < EPT_CHUNKS - 2)
            def _():
                _start_idx(gg + 2, bp)

            @pl.when(gg < EPT_CHUNKS - 1)
            def _():
                _drain_idx(bn)
                _start_gathers(bn)

            _drain_gathers(b)
            for q in range(CHUNK // 16):
                av = asv[b][pl.ds(q * 16, 16)] + adv[b][pl.ds(q * 16, 16)]
                av = jnp.where(av > 0, av, 0.2 * av)
                eev[b][pl.ds(q * 16, 16)] = jnp.exp(av)

            for g2 in range(CHUNK // 16):
                ev = eev[b][pl.ds(g2 * 16, 16)]
                for l in range(16):
                    w = jnp.full((16,), ev[l], jnp.float32)
                    j = g2 * 16 + l
                    for q in range(D // 16):
                        rows[b][j, pl.ds(q * 16, 16)] = (
                            rows[b][j, pl.ds(q * 16, 16)] * w)
            pltpu.async_copy(rows[b], acc.at[didx[b]], sems[b], add=True)
            pltpu.async_copy(eev[b], dacc.at[didx[b]], semd[b], add=True)
        return carry

    lax.fori_loop(0, EPT_CHUNKS // 3, _step, 0)
    _drain_scatter((EPT_CHUNKS - 1) % 3)
    plsc.subcore_barrier()

    @pl.when(c == 0)
    def _():
        for bb in range(nfull):
            r = row0_ + bb * CHUNK
            pltpu.sync_copy(acc.at[pl.ds(r, CHUNK)], out0.at[pl.ds(r, CHUNK)])
        r = row0_ + nfull * CHUNK
        pltpu.sync_copy(acc.at[pl.ds(r, rem)], out0.at[pl.ds(r, rem)])
        pltpu.sync_copy(dacc.at[pl.ds(row0_, ROWS_PER_TILE)],
                        den0.at[pl.ds(row0_, ROWS_PER_TILE)])

    @pl.when(c == 1)
    def _():
        for bb in range(nfull):
            r = row0_ + bb * CHUNK
            pltpu.sync_copy(acc.at[pl.ds(r, CHUNK)], out1.at[pl.ds(r, CHUNK)])
        r = row0_ + nfull * CHUNK
        pltpu.sync_copy(acc.at[pl.ds(r, rem)], out1.at[pl.ds(r, rem)])
        pltpu.sync_copy(dacc.at[pl.ds(row0_, ROWS_PER_TILE)],
                        den1.at[pl.ds(row0_, ROWS_PER_TILE)])


def _edge_pass(src, dst, asrc, adst, h_pad):
    mesh = plsc.VectorSubcoreMesh(core_axis_name="c", subcore_axis_name="s")
    idx_t = pltpu.VMEM((CHUNK,), jnp.int32)
    vec_t = pltpu.VMEM((CHUNK,), jnp.float32)
    row_t = pltpu.VMEM((CHUNK, D), jnp.float32)
    k = functools.partial(
        pl.kernel,
        out_type=[
            jax.ShapeDtypeStruct((NP, D), jnp.float32),
            jax.ShapeDtypeStruct((NP, D), jnp.float32),
            jax.ShapeDtypeStruct((NP,), jnp.float32),
            jax.ShapeDtypeStruct((NP,), jnp.float32),
        ],
        mesh=mesh,
        compiler_params=pltpu.CompilerParams(
            needs_layout_passes=False, use_tc_tiling_on_sc=False),
        scratch_types=(
            [idx_t] * 6 + [vec_t] * 9 + [row_t] * 3
            + [pltpu.VMEM_SHARED((NP, D), jnp.float32),
               pltpu.VMEM_SHARED((NP,), jnp.float32)]
            + [pltpu.SemaphoreType.DMA] * 15
        ),
    )(_edge_body)
    return k(src, dst, asrc, adst, h_pad)


def kernel(x, edge_index, W1, att_src1, att_dst1, bias1,
           W2, att_src2, att_dst2, bias2):
    x_pad = jnp.pad(x, ((0, NP - N), (0, 0)))
    loop = jnp.arange(N, dtype=jnp.int32)
    padi = N + jnp.arange(E_PAD - E - N, dtype=jnp.int32) % (NP - N)
    src = jnp.concatenate([edge_index[0], loop, padi])
    dst = jnp.concatenate([edge_index[1], loop, padi])

    h1, asrc1, adst1 = _prep(x_pad, W1, att_src1, att_dst1)
    p0, p1, d0, d1 = _edge_pass(src, dst, asrc1[:, 0], adst1[:, 0], h1)
    h2, asrc2, adst2 = _combine_prep(p0, p1, d0.reshape(NP, 1),
                                     d1.reshape(NP, 1), bias1.reshape(1, D),
                                     W2, att_src2, att_dst2)
    q0, q1, e0, e1 = _edge_pass(src, dst, asrc2[:, 0], adst2[:, 0], h2)
    out = _combine(q0, q1, e0.reshape(NP, 1), e1.reshape(NP, 1),
                   bias2.reshape(1, D))
    return out[:N]


# speedup vs baseline: 1.3677x; 1.3677x over previous
import functools

import jax
import jax.numpy as jnp
from jax import lax
from jax.experimental import pallas as pl
from jax.experimental.pallas import tpu as pltpu
from jax.experimental.pallas import tpu_sc as plsc

N = 10000
E = 320000
D = 128
NP = 10240
DUMMY = N
NTILES = 32
CHUNK = 112
EPT_CHUNKS = 93
EPT = CHUNK * EPT_CHUNKS
E_PAD = EPT * NTILES
ROWS_PER_TILE = NP // 16


def _prep_body(x_ref, w_ref, as_ref, ad_ref, h_ref, asrc_ref, adst_ref):
    h = lax.dot_general(x_ref[...], w_ref[...], (((1,), (1,)), ((), ())),
                        preferred_element_type=jnp.float32)
    h_ref[...] = h
    asrc_ref[...] = lax.dot_general(h, as_ref[...], (((1,), (1,)), ((), ())),
                                    preferred_element_type=jnp.float32)
    adst_ref[...] = lax.dot_general(h, ad_ref[...], (((1,), (1,)), ((), ())),
                                    preferred_element_type=jnp.float32)


TCB = 1024


def _prep(x_pad, W, a_src, a_dst):
    return pl.pallas_call(
        _prep_body,
        grid=(NP // TCB,),
        in_specs=[
            pl.BlockSpec((TCB, D), lambda i: (i, 0)),
            pl.BlockSpec((D, D), lambda i: (0, 0)),
            pl.BlockSpec((1, D), lambda i: (0, 0)),
            pl.BlockSpec((1, D), lambda i: (0, 0)),
        ],
        out_specs=[
            pl.BlockSpec((TCB, D), lambda i: (i, 0)),
            pl.BlockSpec((TCB, 1), lambda i: (i, 0)),
            pl.BlockSpec((TCB, 1), lambda i: (i, 0)),
        ],
        out_shape=[
            jax.ShapeDtypeStruct((NP, D), jnp.float32),
            jax.ShapeDtypeStruct((NP, 1), jnp.float32),
            jax.ShapeDtypeStruct((NP, 1), jnp.float32),
        ],
    )(x_pad, W, a_src, a_dst)


def _combine_x(p0_ref, p1_ref, d0_ref, d1_ref, b_ref):
    den = d0_ref[...] + d1_ref[...] + 1e-16
    return (p0_ref[...] + p1_ref[...]) / den + b_ref[...]


def _combine_prep_body(p0_ref, p1_ref, d0_ref, d1_ref, b_ref,
                       w_ref, as_ref, ad_ref, h_ref, asrc_ref, adst_ref):
    y = _combine_x(p0_ref, p1_ref, d0_ref, d1_ref, b_ref)
    x2 = jnp.where(y > 0, y, jnp.exp(y) - 1.0)
    h = lax.dot_general(x2, w_ref[...], (((1,), (1,)), ((), ())),
                        preferred_element_type=jnp.float32)
    h_ref[...] = h
    asrc_ref[...] = lax.dot_general(h, as_ref[...], (((1,), (1,)), ((), ())),
                                    preferred_element_type=jnp.float32)
    adst_ref[...] = lax.dot_general(h, ad_ref[...], (((1,), (1,)), ((), ())),
                                    preferred_element_type=jnp.float32)


def _combine_prep(p0, p1, d0, d1, bias, W, a_src, a_dst):
    return pl.pallas_call(
        _combine_prep_body,
        grid=(NP // TCB,),
        in_specs=[
            pl.BlockSpec((TCB, D), lambda i: (i, 0)),
            pl.BlockSpec((TCB, D), lambda i: (i, 0)),
            pl.BlockSpec((TCB, 1), lambda i: (i, 0)),
            pl.BlockSpec((TCB, 1), lambda i: (i, 0)),
            pl.BlockSpec((1, D), lambda i: (0, 0)),
            pl.BlockSpec((D, D), lambda i: (0, 0)),
            pl.BlockSpec((1, D), lambda i: (0, 0)),
            pl.BlockSpec((1, D), lambda i: (0, 0)),
        ],
        out_specs=[
            pl.BlockSpec((TCB, D), lambda i: (i, 0)),
            pl.BlockSpec((TCB, 1), lambda i: (i, 0)),
            pl.BlockSpec((TCB, 1), lambda i: (i, 0)),
        ],
        out_shape=[
            jax.ShapeDtypeStruct((NP, D), jnp.float32),
            jax.ShapeDtypeStruct((NP, 1), jnp.float32),
            jax.ShapeDtypeStruct((NP, 1), jnp.float32),
        ],
    )(p0, p1, d0, d1, bias, W, a_src, a_dst)


def _combine_body(p0_ref, p1_ref, d0_ref, d1_ref, b_ref, y_ref):
    y_ref[...] = _combine_x(p0_ref, p1_ref, d0_ref, d1_ref, b_ref)


def _combine(p0, p1, d0, d1, bias):
    return pl.pallas_call(
        _combine_body,
        grid=(NP // TCB,),
        in_specs=[
            pl.BlockSpec((TCB, D), lambda i: (i, 0)),
            pl.BlockSpec((TCB, D), lambda i: (i, 0)),
            pl.BlockSpec((TCB, 1), lambda i: (i, 0)),
            pl.BlockSpec((TCB, 1), lambda i: (i, 0)),
            pl.BlockSpec((1, D), lambda i: (0, 0)),
        ],
        out_specs=pl.BlockSpec((TCB, D), lambda i: (i, 0)),
        out_shape=jax.ShapeDtypeStruct((NP, D), jnp.float32),
    )(p0, p1, d0, d1, bias)


def _edge_body(src_hbm, dst_hbm, asrc_hbm, adst_hbm, h_hbm,
               out0, out1, den0, den1,
               si0, si1, si2, di0, di1, di2,
               av0, av1, av2, bv0, bv1, bv2,
               ev0, ev1, ev2, r0, r1, r2,
               acc, dacc,
               qi0, qi1, qi2, qa0, qa1, qa2,
               qg0, qg1, qg2, qs0, qs1, qs2, qd0, qd1, qd2):
    sidx = (si0, si1, si2)
    didx = (di0, di1, di2)
    asv = (av0, av1, av2)
    adv = (bv0, bv1, bv2)
    eev = (ev0, ev1, ev2)
    rows = (r0, r1, r2)
    semi = (qi0, qi1, qi2)
    sema = (qa0, qa1, qa2)
    semg = (qg0, qg1, qg2)
    sems = (qs0, qs1, qs2)
    semd = (qd0, qd1, qd2)
    c = lax.axis_index("c")
    s = lax.axis_index("s")
    tile = c * 16 + s
    e0 = tile * EPT

    def _start_idx(g, b):
        base = pl.multiple_of(e0 + g * CHUNK, 8)
        pltpu.async_copy(src_hbm.at[pl.ds(base, CHUNK)], sidx[b], semi[b])
        pltpu.async_copy(dst_hbm.at[pl.ds(base, CHUNK)], didx[b], semi[b])

    def _drain(dummy_hbm, buf, sem):
        pltpu.make_async_copy(dummy_hbm, buf, sem).wait()

    def _drain_idx(b):
        _drain(src_hbm.at[pl.ds(0, CHUNK)], sidx[b], semi[b])
        _drain(src_hbm.at[pl.ds(0, CHUNK)], didx[b], semi[b])

    def _start_gathers(b):
        pltpu.async_copy(h_hbm.at[sidx[b]], rows[b], semg[b])
        pltpu.async_copy(asrc_hbm.at[sidx[b]], asv[b], sema[b])
        pltpu.async_copy(adst_hbm.at[didx[b]], adv[b], sema[b])

    def _drain_gathers(b):
        _drain(h_hbm.at[pl.ds(0, CHUNK)], rows[b], semg[b])
        _drain(asrc_hbm.at[pl.ds(0, CHUNK)], asv[b], sema[b])
        _drain(asrc_hbm.at[pl.ds(0, CHUNK)], adv[b], sema[b])

    def _drain_scatter(b):
        _drain(h_hbm.at[pl.ds(0, CHUNK)], rows[b], sems[b])
        _drain(asrc_hbm.at[pl.ds(0, CHUNK)], eev[b], semd[b])

    z16 = jnp.zeros((16,), jnp.float32)

    def _zrow(i, carry):
        for q in range(D // 16):
            r0[i, pl.ds(q * 16, 16)] = z16
        return carry

    lax.fori_loop(0, CHUNK, _zrow, 0)
    row0_ = s * ROWS_PER_TILE
    nfull = ROWS_PER_TILE // CHUNK
    rem = ROWS_PER_TILE - nfull * CHUNK
    for bb in range(nfull):
        pltpu.sync_copy(r0, acc.at[pl.ds(row0_ + bb * CHUNK, CHUNK)])
    pltpu.sync_copy(r0.at[pl.ds(0, rem)],
                    acc.at[pl.ds(row0_ + nfull * CHUNK, rem)])
    for bb in range(ROWS_PER_TILE // D):
        pltpu.sync_copy(r0.at[0], dacc.at[pl.ds(row0_ + bb * D, D)])

    _start_idx(0, 0)
    _start_idx(1, 1)
    _drain_idx(0)
    _start_gathers(0)
    plsc.subcore_barrier()

    def _step(st, carry):
        for b in range(3):
            gg = st * 3 + b
            bp = (b + 2) % 3
            bn = (b + 1) % 3

            @pl.when(gg >= 1)
            def _():
                _drain_scatter(bp)

            @pl.when(gg < EPT_CHUNKS - 2)
            def _():
                _start_idx(gg + 2, bp)

            @pl.when(gg < EPT_CHUNKS - 1)
            def _():
                _drain_idx(bn)
                _start_gathers(bn)

            _drain_gathers(b)
            for q in range(CHUNK // 16):
                av = asv[b][pl.ds(q * 16, 16)] + adv[b][pl.ds(q * 16, 16)]
                av = jnp.where(av > 0, av, 0.2 * av)
                eev[b][pl.ds(q * 16, 16)] = jnp.exp(av)

            @plsc.parallel_loop(0, CHUNK // 16, unroll=2)
            def _scale(g2):
                off = pl.multiple_of(g2 * 16, 16)
                ev = eev[b][pl.ds(off, 16)]
                for l in range(16):
                    w = jnp.full((16,), ev[l], jnp.float32)
                    for q in range(D // 16):
                        rows[b][off + l, pl.ds(q * 16, 16)] = (
                            rows[b][off + l, pl.ds(q * 16, 16)] * w)
            pltpu.async_copy(rows[b], acc.at[didx[b]], sems[b], add=True)
            pltpu.async_copy(eev[b], dacc.at[didx[b]], semd[b], add=True)
        return carry

    lax.fori_loop(0, EPT_CHUNKS // 3, _step, 0)
    _drain_scatter((EPT_CHUNKS - 1) % 3)
    plsc.subcore_barrier()

    @pl.when(c == 0)
    def _():
        for bb in range(nfull):
            r = row0_ + bb * CHUNK
            pltpu.sync_copy(acc.at[pl.ds(r, CHUNK)], out0.at[pl.ds(r, CHUNK)])
        r = row0_ + nfull * CHUNK
        pltpu.sync_copy(acc.at[pl.ds(r, rem)], out0.at[pl.ds(r, rem)])
        pltpu.sync_copy(dacc.at[pl.ds(row0_, ROWS_PER_TILE)],
                        den0.at[pl.ds(row0_, ROWS_PER_TILE)])

    @pl.when(c == 1)
    def _():
        for bb in range(nfull):
            r = row0_ + bb * CHUNK
            pltpu.sync_copy(acc.at[pl.ds(r, CHUNK)], out1.at[pl.ds(r, CHUNK)])
        r = row0_ + nfull * CHUNK
        pltpu.sync_copy(acc.at[pl.ds(r, rem)], out1.at[pl.ds(r, rem)])
        pltpu.sync_copy(dacc.at[pl.ds(row0_, ROWS_PER_TILE)],
                        den1.at[pl.ds(row0_, ROWS_PER_TILE)])


def _edge_pass(src, dst, asrc, adst, h_pad):
    mesh = plsc.VectorSubcoreMesh(core_axis_name="c", subcore_axis_name="s")
    idx_t = pltpu.VMEM((CHUNK,), jnp.int32)
    vec_t = pltpu.VMEM((CHUNK,), jnp.float32)
    row_t = pltpu.VMEM((CHUNK, D), jnp.float32)
    k = functools.partial(
        pl.kernel,
        out_type=[
            jax.ShapeDtypeStruct((NP, D), jnp.float32),
            jax.ShapeDtypeStruct((NP, D), jnp.float32),
            jax.ShapeDtypeStruct((NP,), jnp.float32),
            jax.ShapeDtypeStruct((NP,), jnp.float32),
        ],
        mesh=mesh,
        compiler_params=pltpu.CompilerParams(
            needs_layout_passes=False, use_tc_tiling_on_sc=False),
        scratch_types=(
            [idx_t] * 6 + [vec_t] * 9 + [row_t] * 3
            + [pltpu.VMEM_SHARED((NP, D), jnp.float32),
               pltpu.VMEM_SHARED((NP,), jnp.float32)]
            + [pltpu.SemaphoreType.DMA] * 15
        ),
    )(_edge_body)
    return k(src, dst, asrc, adst, h_pad)


def kernel(x, edge_index, W1, att_src1, att_dst1, bias1,
           W2, att_src2, att_dst2, bias2):
    x_pad = jnp.pad(x, ((0, NP - N), (0, 0)))
    loop = jnp.arange(N, dtype=jnp.int32)
    padi = N + jnp.arange(E_PAD - E - N, dtype=jnp.int32) % (NP - N)
    src = jnp.concatenate([edge_index[0], loop, padi])
    dst = jnp.concatenate([edge_index[1], loop, padi])

    h1, asrc1, adst1 = _prep(x_pad, W1, att_src1, att_dst1)
    p0, p1, d0, d1 = _edge_pass(src, dst, asrc1[:, 0], adst1[:, 0], h1)
    h2, asrc2, adst2 = _combine_prep(p0, p1, d0.reshape(NP, 1),
                                     d1.reshape(NP, 1), bias1.reshape(1, D),
                                     W2, att_src2, att_dst2)
    q0, q1, e0, e1 = _edge_pass(src, dst, asrc2[:, 0], adst2[:, 0], h2)
    out = _combine(q0, q1, e0.reshape(NP, 1), e1.reshape(NP, 1),
                   bias2.reshape(1, D))
    return out[:N]


# speedup vs baseline: 1.4545x; 1.0635x over previous
import functools

import jax
import jax.numpy as jnp
from jax import lax
from jax.experimental import pallas as pl
from jax.experimental.pallas import tpu as pltpu
from jax.experimental.pallas import tpu_sc as plsc

N = 10000
E = 320000
D = 128
NP = 10240
DUMMY = N
NTILES = 32
CHUNK = 112
EPT_CHUNKS = 93
EPT = CHUNK * EPT_CHUNKS
E_PAD = EPT * NTILES
ROWS_PER_TILE = NP // 16


def _prep_body(x_ref, w_ref, as_ref, ad_ref, h_ref, asrc_ref, adst_ref):
    h = lax.dot_general(x_ref[...], w_ref[...], (((1,), (1,)), ((), ())),
                        preferred_element_type=jnp.float32)
    h_ref[...] = h
    asrc_ref[...] = lax.dot_general(h, as_ref[...], (((1,), (1,)), ((), ())),
                                    preferred_element_type=jnp.float32)
    adst_ref[...] = lax.dot_general(h, ad_ref[...], (((1,), (1,)), ((), ())),
                                    preferred_element_type=jnp.float32)


TCB = 1024


def _prep(x_pad, W, a_src, a_dst):
    return pl.pallas_call(
        _prep_body,
        grid=(NP // TCB,),
        in_specs=[
            pl.BlockSpec((TCB, D), lambda i: (i, 0)),
            pl.BlockSpec((D, D), lambda i: (0, 0)),
            pl.BlockSpec((1, D), lambda i: (0, 0)),
            pl.BlockSpec((1, D), lambda i: (0, 0)),
        ],
        out_specs=[
            pl.BlockSpec((TCB, D), lambda i: (i, 0)),
            pl.BlockSpec((TCB, 1), lambda i: (i, 0)),
            pl.BlockSpec((TCB, 1), lambda i: (i, 0)),
        ],
        out_shape=[
            jax.ShapeDtypeStruct((NP, D), jnp.float32),
            jax.ShapeDtypeStruct((NP, 1), jnp.float32),
            jax.ShapeDtypeStruct((NP, 1), jnp.float32),
        ],
    )(x_pad, W, a_src, a_dst)


def _combine_x(p0_ref, p1_ref, d0_ref, d1_ref, b_ref):
    den = d0_ref[...] + d1_ref[...] + 1e-16
    return (p0_ref[...] + p1_ref[...]) / den + b_ref[...]


def _combine_prep_body(p0_ref, p1_ref, d0_ref, d1_ref, b_ref,
                       w_ref, as_ref, ad_ref, h_ref, asrc_ref, adst_ref):
    y = _combine_x(p0_ref, p1_ref, d0_ref, d1_ref, b_ref)
    x2 = jnp.where(y > 0, y, jnp.exp(y) - 1.0)
    h = lax.dot_general(x2, w_ref[...], (((1,), (1,)), ((), ())),
                        preferred_element_type=jnp.float32)
    h_ref[...] = h
    asrc_ref[...] = lax.dot_general(h, as_ref[...], (((1,), (1,)), ((), ())),
                                    preferred_element_type=jnp.float32)
    adst_ref[...] = lax.dot_general(h, ad_ref[...], (((1,), (1,)), ((), ())),
                                    preferred_element_type=jnp.float32)


def _combine_prep(p0, p1, d0, d1, bias, W, a_src, a_dst):
    return pl.pallas_call(
        _combine_prep_body,
        grid=(NP // TCB,),
        in_specs=[
            pl.BlockSpec((TCB, D), lambda i: (i, 0)),
            pl.BlockSpec((TCB, D), lambda i: (i, 0)),
            pl.BlockSpec((TCB, 1), lambda i: (i, 0)),
            pl.BlockSpec((TCB, 1), lambda i: (i, 0)),
            pl.BlockSpec((1, D), lambda i: (0, 0)),
            pl.BlockSpec((D, D), lambda i: (0, 0)),
            pl.BlockSpec((1, D), lambda i: (0, 0)),
            pl.BlockSpec((1, D), lambda i: (0, 0)),
        ],
        out_specs=[
            pl.BlockSpec((TCB, D), lambda i: (i, 0)),
            pl.BlockSpec((TCB, 1), lambda i: (i, 0)),
            pl.BlockSpec((TCB, 1), lambda i: (i, 0)),
        ],
        out_shape=[
            jax.ShapeDtypeStruct((NP, D), jnp.float32),
            jax.ShapeDtypeStruct((NP, 1), jnp.float32),
            jax.ShapeDtypeStruct((NP, 1), jnp.float32),
        ],
    )(p0, p1, d0, d1, bias, W, a_src, a_dst)


def _combine_body(p0_ref, p1_ref, d0_ref, d1_ref, b_ref, y_ref):
    y_ref[...] = _combine_x(p0_ref, p1_ref, d0_ref, d1_ref, b_ref)


def _combine(p0, p1, d0, d1, bias):
    return pl.pallas_call(
        _combine_body,
        grid=(NP // TCB,),
        in_specs=[
            pl.BlockSpec((TCB, D), lambda i: (i, 0)),
            pl.BlockSpec((TCB, D), lambda i: (i, 0)),
            pl.BlockSpec((TCB, 1), lambda i: (i, 0)),
            pl.BlockSpec((TCB, 1), lambda i: (i, 0)),
            pl.BlockSpec((1, D), lambda i: (0, 0)),
        ],
        out_specs=pl.BlockSpec((TCB, D), lambda i: (i, 0)),
        out_shape=jax.ShapeDtypeStruct((NP, D), jnp.float32),
    )(p0, p1, d0, d1, bias)


def _edge_body(ei_hbm, asrc_hbm, adst_hbm, h_hbm,
               out0, out1, den0, den1,
               ix0, ix1, ix2,
               av0, av1, av2, bv0, bv1, bv2,
               ev0, ev1, ev2, r0, r1, r2,
               acc, dacc,
               qi0, qi1, qi2, qa0, qa1, qa2,
               qg0, qg1, qg2, qs0, qs1, qs2, qd0, qd1, qd2):
    idx2 = (ix0, ix1, ix2)
    asv = (av0, av1, av2)
    adv = (bv0, bv1, bv2)
    eev = (ev0, ev1, ev2)
    rows = (r0, r1, r2)
    semi = (qi0, qi1, qi2)
    sema = (qa0, qa1, qa2)
    semg = (qg0, qg1, qg2)
    sems = (qs0, qs1, qs2)
    semd = (qd0, qd1, qd2)
    c = lax.axis_index("c")
    s = lax.axis_index("s")
    tile = c * 16 + s
    e0 = tile * EPT

    def _start_idx(g, b):
        base = pl.multiple_of(e0 + g * CHUNK, 8)
        pltpu.async_copy(ei_hbm.at[:, pl.ds(base, CHUNK)], idx2[b], semi[b])

    def _drain(dummy_hbm, buf, sem):
        pltpu.make_async_copy(dummy_hbm, buf, sem).wait()

    def _drain_idx(b):
        _drain(ei_hbm.at[:, pl.ds(0, CHUNK)], idx2[b], semi[b])

    def _start_gathers(b):
        pltpu.async_copy(h_hbm.at[idx2[b].at[0]], rows[b], semg[b])
        pltpu.async_copy(asrc_hbm.at[idx2[b].at[0]], asv[b], sema[b])
        pltpu.async_copy(adst_hbm.at[idx2[b].at[1]], adv[b], sema[b])

    def _drain_gathers(b):
        _drain(h_hbm.at[pl.ds(0, CHUNK)], rows[b], semg[b])
        _drain(asrc_hbm.at[pl.ds(0, CHUNK)], asv[b], sema[b])
        _drain(asrc_hbm.at[pl.ds(0, CHUNK)], adv[b], sema[b])

    def _drain_scatter(b):
        _drain(h_hbm.at[pl.ds(0, CHUNK)], rows[b], sems[b])
        _drain(asrc_hbm.at[pl.ds(0, CHUNK)], eev[b], semd[b])

    z16 = jnp.zeros((16,), jnp.float32)

    def _zrow(i, carry):
        for q in range(D // 16):
            r0[i, pl.ds(q * 16, 16)] = z16
        return carry

    lax.fori_loop(0, CHUNK, _zrow, 0)
    row0_ = s * ROWS_PER_TILE
    nfull = ROWS_PER_TILE // CHUNK
    rem = ROWS_PER_TILE - nfull * CHUNK
    for bb in range(nfull):
        pltpu.sync_copy(r0, acc.at[pl.ds(row0_ + bb * CHUNK, CHUNK)])
    pltpu.sync_copy(r0.at[pl.ds(0, rem)],
                    acc.at[pl.ds(row0_ + nfull * CHUNK, rem)])
    for bb in range(ROWS_PER_TILE // D):
        pltpu.sync_copy(r0.at[0], dacc.at[pl.ds(row0_ + bb * D, D)])

    _start_idx(0, 0)
    _start_idx(1, 1)
    _drain_idx(0)
    _start_gathers(0)
    plsc.subcore_barrier()

    def _step(st, carry):
        for b in range(3):
            gg = st * 3 + b
            bp = (b + 2) % 3
            bn = (b + 1) % 3

            @pl.when(gg >= 1)
            def _():
                _drain_scatter(bp)

            @pl.when(gg < EPT_CHUNKS - 2)
            def _():
                _start_idx(gg + 2, bp)

            @pl.when(gg < EPT_CHUNKS - 1)
            def _():
                _drain_idx(bn)
                _start_gathers(bn)

            _drain_gathers(b)
            for q in range(CHUNK // 16):
                av = asv[b][pl.ds(q * 16, 16)] + adv[b][pl.ds(q * 16, 16)]
                av = jnp.where(av > 0, av, 0.2 * av)
                eev[b][pl.ds(q * 16, 16)] = jnp.exp(av)

            @plsc.parallel_loop(0, CHUNK // 16, unroll=2)
            def _scale(g2):
                off = pl.multiple_of(g2 * 16, 16)
                ev = eev[b][pl.ds(off, 16)]
                for l in range(16):
                    w = jnp.full((16,), ev[l], jnp.float32)
                    for q in range(D // 16):
                        rows[b][off + l, pl.ds(q * 16, 16)] = (
                            rows[b][off + l, pl.ds(q * 16, 16)] * w)
            pltpu.async_copy(rows[b], acc.at[idx2[b].at[1]], sems[b],
                             add=True)
            pltpu.async_copy(eev[b], dacc.at[idx2[b].at[1]], semd[b],
                             add=True)
        return carry

    lax.fori_loop(0, EPT_CHUNKS // 3, _step, 0)
    _drain_scatter((EPT_CHUNKS - 1) % 3)
    plsc.subcore_barrier()

    @pl.when(c == 0)
    def _():
        for bb in range(nfull):
            r = row0_ + bb * CHUNK
            pltpu.sync_copy(acc.at[pl.ds(r, CHUNK)], out0.at[pl.ds(r, CHUNK)])
        r = row0_ + nfull * CHUNK
        pltpu.sync_copy(acc.at[pl.ds(r, rem)], out0.at[pl.ds(r, rem)])
        pltpu.sync_copy(dacc.at[pl.ds(row0_, ROWS_PER_TILE)],
                        den0.at[pl.ds(row0_, ROWS_PER_TILE)])

    @pl.when(c == 1)
    def _():
        for bb in range(nfull):
            r = row0_ + bb * CHUNK
            pltpu.sync_copy(acc.at[pl.ds(r, CHUNK)], out1.at[pl.ds(r, CHUNK)])
        r = row0_ + nfull * CHUNK
        pltpu.sync_copy(acc.at[pl.ds(r, rem)], out1.at[pl.ds(r, rem)])
        pltpu.sync_copy(dacc.at[pl.ds(row0_, ROWS_PER_TILE)],
                        den1.at[pl.ds(row0_, ROWS_PER_TILE)])


def _edge_pass(src_dst, asrc, adst, h_pad):
    mesh = plsc.VectorSubcoreMesh(core_axis_name="c", subcore_axis_name="s")
    idx_t = pltpu.VMEM((2, CHUNK), jnp.int32)
    vec_t = pltpu.VMEM((CHUNK,), jnp.float32)
    row_t = pltpu.VMEM((CHUNK, D), jnp.float32)
    k = functools.partial(
        pl.kernel,
        out_type=[
            jax.ShapeDtypeStruct((NP, D), jnp.float32),
            jax.ShapeDtypeStruct((NP, D), jnp.float32),
            jax.ShapeDtypeStruct((NP,), jnp.float32),
            jax.ShapeDtypeStruct((NP,), jnp.float32),
        ],
        mesh=mesh,
        compiler_params=pltpu.CompilerParams(
            needs_layout_passes=False, use_tc_tiling_on_sc=False),
        scratch_types=(
            [idx_t] * 3 + [vec_t] * 9 + [row_t] * 3
            + [pltpu.VMEM_SHARED((NP, D), jnp.float32),
               pltpu.VMEM_SHARED((NP,), jnp.float32)]
            + [pltpu.SemaphoreType.DMA] * 15
        ),
    )(_edge_body)
    return k(src_dst, asrc, adst, h_pad)


def kernel(x, edge_index, W1, att_src1, att_dst1, bias1,
           W2, att_src2, att_dst2, bias2):
    x_pad = jnp.pad(x, ((0, NP - N), (0, 0)))
    loop = jnp.arange(N, dtype=jnp.int32)
    padi = N + jnp.arange(E_PAD - E - N, dtype=jnp.int32) % (NP - N)
    tail = jnp.stack([jnp.concatenate([loop, padi])] * 2)
    src_dst = jnp.concatenate([edge_index, tail], axis=1)

    h1, asrc1, adst1 = _prep(x_pad, W1, att_src1, att_dst1)
    p0, p1, d0, d1 = _edge_pass(src_dst, asrc1[:, 0], adst1[:, 0], h1)
    h2, asrc2, adst2 = _combine_prep(p0, p1, d0.reshape(NP, 1),
                                     d1.reshape(NP, 1), bias1.reshape(1, D),
                                     W2, att_src2, att_dst2)
    q0, q1, e0, e1 = _edge_pass(src_dst, asrc2[:, 0], adst2[:, 0], h2)
    out = _combine(q0, q1, e0.reshape(NP, 1), e1.reshape(NP, 1),
                   bias2.reshape(1, D))
    return out[:N]


# speedup vs baseline: 1.5176x; 1.0434x over previous
import functools

import jax
import jax.numpy as jnp
from jax import lax
from jax.experimental import pallas as pl
from jax.experimental.pallas import tpu as pltpu
from jax.experimental.pallas import tpu_sc as plsc

N = 10000
E = 320000
D = 128
NP = 10240
DUMMY = N
NTILES = 32
CHUNK = 112
EPT_CHUNKS = 93
EPT = CHUNK * EPT_CHUNKS
E_PAD = EPT * NTILES
ROWS_PER_TILE = NP // 16


def _prep_body(x_ref, w_ref, as_ref, ad_ref, h_ref, asrc_ref, adst_ref):
    h = lax.dot_general(x_ref[...], w_ref[...], (((1,), (1,)), ((), ())),
                        preferred_element_type=jnp.float32)
    h_ref[...] = h
    asrc_ref[...] = lax.dot_general(as_ref[...], h, (((1,), (1,)), ((), ())),
                                    preferred_element_type=jnp.float32)
    adst_ref[...] = lax.dot_general(ad_ref[...], h, (((1,), (1,)), ((), ())),
                                    preferred_element_type=jnp.float32)


TCB = 1024


def _prep(x_pad, W, a_src, a_dst):
    return pl.pallas_call(
        _prep_body,
        grid=(NP // TCB,),
        in_specs=[
            pl.BlockSpec((TCB, D), lambda i: (i, 0)),
            pl.BlockSpec((D, D), lambda i: (0, 0)),
            pl.BlockSpec((1, D), lambda i: (0, 0)),
            pl.BlockSpec((1, D), lambda i: (0, 0)),
        ],
        out_specs=[
            pl.BlockSpec((TCB, D), lambda i: (i, 0)),
            pl.BlockSpec((1, TCB), lambda i: (0, i)),
            pl.BlockSpec((1, TCB), lambda i: (0, i)),
        ],
        out_shape=[
            jax.ShapeDtypeStruct((NP, D), jnp.float32),
            jax.ShapeDtypeStruct((1, NP), jnp.float32),
            jax.ShapeDtypeStruct((1, NP), jnp.float32),
        ],
    )(x_pad, W, a_src, a_dst)


def _combine_x(p0_ref, p1_ref, d0_ref, d1_ref, b_ref):
    den = d0_ref[...] + d1_ref[...] + 1e-16
    return (p0_ref[...] + p1_ref[...]) / den + b_ref[...]


def _combine_prep_body(p0_ref, p1_ref, d0_ref, d1_ref, b_ref,
                       w_ref, as_ref, ad_ref, h_ref, asrc_ref, adst_ref):
    y = _combine_x(p0_ref, p1_ref, d0_ref, d1_ref, b_ref)
    x2 = jnp.where(y > 0, y, jnp.exp(y) - 1.0)
    h = lax.dot_general(x2, w_ref[...], (((1,), (1,)), ((), ())),
                        preferred_element_type=jnp.float32)
    h_ref[...] = h
    asrc_ref[...] = lax.dot_general(as_ref[...], h, (((1,), (1,)), ((), ())),
                                    preferred_element_type=jnp.float32)
    adst_ref[...] = lax.dot_general(ad_ref[...], h, (((1,), (1,)), ((), ())),
                                    preferred_element_type=jnp.float32)


def _combine_prep(p0, p1, d0, d1, bias, W, a_src, a_dst):
    return pl.pallas_call(
        _combine_prep_body,
        grid=(NP // TCB,),
        in_specs=[
            pl.BlockSpec((TCB, D), lambda i: (i, 0)),
            pl.BlockSpec((TCB, D), lambda i: (i, 0)),
            pl.BlockSpec((TCB, 1), lambda i: (i, 0)),
            pl.BlockSpec((TCB, 1), lambda i: (i, 0)),
            pl.BlockSpec((1, D), lambda i: (0, 0)),
            pl.BlockSpec((D, D), lambda i: (0, 0)),
            pl.BlockSpec((1, D), lambda i: (0, 0)),
            pl.BlockSpec((1, D), lambda i: (0, 0)),
        ],
        out_specs=[
            pl.BlockSpec((TCB, D), lambda i: (i, 0)),
            pl.BlockSpec((1, TCB), lambda i: (0, i)),
            pl.BlockSpec((1, TCB), lambda i: (0, i)),
        ],
        out_shape=[
            jax.ShapeDtypeStruct((NP, D), jnp.float32),
            jax.ShapeDtypeStruct((1, NP), jnp.float32),
            jax.ShapeDtypeStruct((1, NP), jnp.float32),
        ],
    )(p0, p1, d0, d1, bias, W, a_src, a_dst)


def _combine_body(p0_ref, p1_ref, d0_ref, d1_ref, b_ref, y_ref):
    y_ref[...] = _combine_x(p0_ref, p1_ref, d0_ref, d1_ref, b_ref)


def _combine(p0, p1, d0, d1, bias):
    return pl.pallas_call(
        _combine_body,
        grid=(NP // TCB,),
        in_specs=[
            pl.BlockSpec((TCB, D), lambda i: (i, 0)),
            pl.BlockSpec((TCB, D), lambda i: (i, 0)),
            pl.BlockSpec((TCB, 1), lambda i: (i, 0)),
            pl.BlockSpec((TCB, 1), lambda i: (i, 0)),
            pl.BlockSpec((1, D), lambda i: (0, 0)),
        ],
        out_specs=pl.BlockSpec((TCB, D), lambda i: (i, 0)),
        out_shape=jax.ShapeDtypeStruct((NP, D), jnp.float32),
    )(p0, p1, d0, d1, bias)


def _edge_body(ei_hbm, asrc_hbm, adst_hbm, h_hbm,
               out0, out1, den0, den1,
               ix0, ix1, ix2,
               av0, av1, av2, bv0, bv1, bv2,
               ev0, ev1, ev2, r0, r1, r2,
               acc, dacc,
               qi0, qi1, qi2, qa0, qa1, qa2,
               qg0, qg1, qg2, qs0, qs1, qs2, qd0, qd1, qd2):
    idx2 = (ix0, ix1, ix2)
    asv = (av0, av1, av2)
    adv = (bv0, bv1, bv2)
    eev = (ev0, ev1, ev2)
    rows = (r0, r1, r2)
    semi = (qi0, qi1, qi2)
    sema = (qa0, qa1, qa2)
    semg = (qg0, qg1, qg2)
    sems = (qs0, qs1, qs2)
    semd = (qd0, qd1, qd2)
    c = lax.axis_index("c")
    s = lax.axis_index("s")
    tile = c * 16 + s
    e0 = tile * EPT

    def _start_idx(g, b):
        base = pl.multiple_of(e0 + g * CHUNK, 8)
        pltpu.async_copy(ei_hbm.at[:, pl.ds(base, CHUNK)], idx2[b], semi[b])

    def _drain(dummy_hbm, buf, sem):
        pltpu.make_async_copy(dummy_hbm, buf, sem).wait()

    def _drain_idx(b):
        _drain(ei_hbm.at[:, pl.ds(0, CHUNK)], idx2[b], semi[b])

    def _start_gathers(b):
        pltpu.async_copy(h_hbm.at[idx2[b].at[0]], rows[b], semg[b])
        pltpu.async_copy(asrc_hbm.at[idx2[b].at[0]], asv[b], sema[b])
        pltpu.async_copy(adst_hbm.at[idx2[b].at[1]], adv[b], sema[b])

    def _drain_gathers(b):
        _drain(h_hbm.at[pl.ds(0, CHUNK)], rows[b], semg[b])
        _drain(asrc_hbm.at[pl.ds(0, CHUNK)], asv[b], sema[b])
        _drain(asrc_hbm.at[pl.ds(0, CHUNK)], adv[b], sema[b])

    def _drain_scatter(b):
        _drain(h_hbm.at[pl.ds(0, CHUNK)], rows[b], sems[b])
        _drain(asrc_hbm.at[pl.ds(0, CHUNK)], eev[b], semd[b])

    z16 = jnp.zeros((16,), jnp.float32)

    def _zrow(i, carry):
        for q in range(D // 16):
            r0[i, pl.ds(q * 16, 16)] = z16
        return carry

    lax.fori_loop(0, CHUNK, _zrow, 0)
    row0_ = s * ROWS_PER_TILE
    nfull = ROWS_PER_TILE // CHUNK
    rem = ROWS_PER_TILE - nfull * CHUNK
    for bb in range(nfull):
        pltpu.sync_copy(r0, acc.at[pl.ds(row0_ + bb * CHUNK, CHUNK)])
    pltpu.sync_copy(r0.at[pl.ds(0, rem)],
                    acc.at[pl.ds(row0_ + nfull * CHUNK, rem)])
    for bb in range(ROWS_PER_TILE // D):
        pltpu.sync_copy(r0.at[0], dacc.at[pl.ds(row0_ + bb * D, D)])

    _start_idx(0, 0)
    _start_idx(1, 1)
    _drain_idx(0)
    _start_gathers(0)
    plsc.subcore_barrier()

    def _step(st, carry):
        for b in range(3):
            gg = st * 3 + b
            bp = (b + 2) % 3
            bn = (b + 1) % 3

            @pl.when(gg >= 1)
            def _():
                _drain_scatter(bp)

            @pl.when(gg < EPT_CHUNKS - 2)
            def _():
                _start_idx(gg + 2, bp)

            @pl.when(gg < EPT_CHUNKS - 1)
            def _():
                _drain_idx(bn)
                _start_gathers(bn)

            _drain_gathers(b)
            for q in range(CHUNK // 16):
                av = asv[b][pl.ds(q * 16, 16)] + adv[b][pl.ds(q * 16, 16)]
                av = jnp.where(av > 0, av, 0.2 * av)
                eev[b][pl.ds(q * 16, 16)] = jnp.exp(av)

            @plsc.parallel_loop(0, CHUNK // 16, unroll=2)
            def _scale(g2):
                off = pl.multiple_of(g2 * 16, 16)
                ev = eev[b][pl.ds(off, 16)]
                for l in range(16):
                    w = jnp.full((16,), ev[l], jnp.float32)
                    for q in range(D // 16):
                        rows[b][off + l, pl.ds(q * 16, 16)] = (
                            rows[b][off + l, pl.ds(q * 16, 16)] * w)
            pltpu.async_copy(rows[b], acc.at[idx2[b].at[1]], sems[b],
                             add=True)
            pltpu.async_copy(eev[b], dacc.at[idx2[b].at[1]], semd[b],
                             add=True)
        return carry

    lax.fori_loop(0, EPT_CHUNKS // 3, _step, 0)
    _drain_scatter((EPT_CHUNKS - 1) % 3)
    plsc.subcore_barrier()

    @pl.when(c == 0)
    def _():
        for bb in range(nfull):
            r = row0_ + bb * CHUNK
            pltpu.sync_copy(acc.at[pl.ds(r, CHUNK)], out0.at[pl.ds(r, CHUNK)])
        r = row0_ + nfull * CHUNK
        pltpu.sync_copy(acc.at[pl.ds(r, rem)], out0.at[pl.ds(r, rem)])
        pltpu.sync_copy(dacc.at[pl.ds(row0_, ROWS_PER_TILE)],
                        den0.at[pl.ds(row0_, ROWS_PER_TILE)])

    @pl.when(c == 1)
    def _():
        for bb in range(nfull):
            r = row0_ + bb * CHUNK
            pltpu.sync_copy(acc.at[pl.ds(r, CHUNK)], out1.at[pl.ds(r, CHUNK)])
        r = row0_ + nfull * CHUNK
        pltpu.sync_copy(acc.at[pl.ds(r, rem)], out1.at[pl.ds(r, rem)])
        pltpu.sync_copy(dacc.at[pl.ds(row0_, ROWS_PER_TILE)],
                        den1.at[pl.ds(row0_, ROWS_PER_TILE)])


def _edge_pass(src_dst, asrc, adst, h_pad):
    mesh = plsc.VectorSubcoreMesh(core_axis_name="c", subcore_axis_name="s")
    idx_t = pltpu.VMEM((2, CHUNK), jnp.int32)
    vec_t = pltpu.VMEM((CHUNK,), jnp.float32)
    row_t = pltpu.VMEM((CHUNK, D), jnp.float32)
    k = functools.partial(
        pl.kernel,
        out_type=[
            jax.ShapeDtypeStruct((NP, D), jnp.float32),
            jax.ShapeDtypeStruct((NP, D), jnp.float32),
            jax.ShapeDtypeStruct((NP,), jnp.float32),
            jax.ShapeDtypeStruct((NP,), jnp.float32),
        ],
        mesh=mesh,
        compiler_params=pltpu.CompilerParams(
            needs_layout_passes=False, use_tc_tiling_on_sc=False),
        scratch_types=(
            [idx_t] * 3 + [vec_t] * 9 + [row_t] * 3
            + [pltpu.VMEM_SHARED((NP, D), jnp.float32),
               pltpu.VMEM_SHARED((NP,), jnp.float32)]
            + [pltpu.SemaphoreType.DMA] * 15
        ),
    )(_edge_body)
    return k(src_dst, asrc, adst, h_pad)


def kernel(x, edge_index, W1, att_src1, att_dst1, bias1,
           W2, att_src2, att_dst2, bias2):
    x_pad = jnp.pad(x, ((0, NP - N), (0, 0)))
    loop = jnp.arange(N, dtype=jnp.int32)
    padi = N + jnp.arange(E_PAD - E - N, dtype=jnp.int32) % (NP - N)
    tail = jnp.stack([jnp.concatenate([loop, padi])] * 2)
    src_dst = jnp.concatenate([edge_index, tail], axis=1)

    h1, asrc1, adst1 = _prep(x_pad, W1, att_src1, att_dst1)
    p0, p1, d0, d1 = _edge_pass(src_dst, asrc1.reshape(NP),
                                adst1.reshape(NP), h1)
    h2, asrc2, adst2 = _combine_prep(p0, p1, d0.reshape(NP, 1),
                                     d1.reshape(NP, 1), bias1.reshape(1, D),
                                     W2, att_src2, att_dst2)
    q0, q1, e0, e1 = _edge_pass(src_dst, asrc2.reshape(NP),
                                adst2.reshape(NP), h2)
    out = _combine(q0, q1, e0.reshape(NP, 1), e1.reshape(NP, 1),
                   bias2.reshape(1, D))
    return out[:N]
